# baseline (speedup 1.0000x reference)
import functools
import jax
import jax.numpy as jnp
from jax import lax
from jax.experimental import pallas as pl
from jax.experimental.pallas import tpu as pltpu
from jax.experimental.pallas import tpu_sc as plsc

HIDDEN = 128


def _pos_body(out_ref):
    L, H = out_ref.shape
    half = H // 2
    pos = lax.broadcasted_iota(jnp.int32, (L, H), 0).astype(jnp.float32)
    col = lax.broadcasted_iota(jnp.int32, (L, H), 1)
    k = jnp.where(col < half, col, col - half).astype(jnp.float32)
    inv_freq = jnp.exp(k * (-2.0 / H * 9.210340371976184))
    angle = pos * inv_freq
    out_ref[...] = jnp.where(col < half, jnp.sin(angle), jnp.cos(angle))


def _make_pos(L, H):
    return pl.pallas_call(
        _pos_body,
        out_shape=jax.ShapeDtypeStruct((L, H), jnp.float32),
    )()


def _sc_body(P, L, H, idx_hbm, table_hbm, pos_hbm, out_hbm,
             idx_v, rows_v, pos_v, sem):
    nc = 2
    wid = lax.axis_index("s") * nc + lax.axis_index("c")
    base = wid * (P * L)
    pltpu.sync_copy(pos_hbm, pos_v)

    def period(p, carry):
        off = base + p * L
        pltpu.sync_copy(idx_hbm.at[pl.ds(off, L)], idx_v)
        pltpu.async_copy(table_hbm.at[idx_v], rows_v, sem).wait()

        def add_row(l, c):
            for h in range(H // 16):
                s = pl.ds(h * 16, 16)
                rows_v[l, s] = rows_v[l, s] + pos_v[l, s]
            return c

        lax.fori_loop(0, L, add_row, 0)
        pltpu.sync_copy(rows_v, out_hbm.at[pl.ds(off, L)])
        return carry

    lax.fori_loop(0, P, period, 0)


def kernel(tokens, emb_table):
    B, L = tokens.shape
    V, H = emb_table.shape
    NW = 32
    N = B * L
    assert N % NW == 0 and (N // NW) % L == 0 and H % 16 == 0
    P = (N // NW) // L

    pos = _make_pos(L, H)
    idx_flat = tokens.reshape(N).astype(jnp.int32)

    mesh = plsc.VectorSubcoreMesh(core_axis_name="c", subcore_axis_name="s")
    sc_k = functools.partial(
        pl.kernel,
        out_type=jax.ShapeDtypeStruct((N, H), jnp.float32),
        mesh=mesh,
        scratch_types=[
            pltpu.VMEM((L,), jnp.int32),
            pltpu.VMEM((L, H), jnp.float32),
            pltpu.VMEM((L, H), jnp.float32),
            pltpu.SemaphoreType.DMA,
        ],
    )(functools.partial(_sc_body, P, L, H))

    out = sc_k(idx_flat, emb_table, pos)
    return out.reshape(B, L, H)


# speedup vs baseline: 1.7583x; 1.7583x over previous
import functools
import jax
import jax.numpy as jnp
from jax import lax
from jax.experimental import pallas as pl
from jax.experimental.pallas import tpu as pltpu
from jax.experimental.pallas import tpu_sc as plsc

HIDDEN = 128


def _pos_body(out_ref):
    L, H = out_ref.shape
    half = H // 2
    pos = lax.broadcasted_iota(jnp.int32, (L, H), 0).astype(jnp.float32)
    col = lax.broadcasted_iota(jnp.int32, (L, H), 1)
    k = jnp.where(col < half, col, col - half).astype(jnp.float32)
    inv_freq = jnp.exp(k * (-2.0 / H * 9.210340371976184))
    angle = pos * inv_freq
    out_ref[...] = jnp.where(col < half, jnp.sin(angle), jnp.cos(angle))


def _make_pos(L, H):
    return pl.pallas_call(
        _pos_body,
        out_shape=jax.ShapeDtypeStruct((L, H), jnp.float32),
    )()


def _sc_body(P, L, H, idx_hbm, table_hbm, pos_hbm, out_hbm,
             idx_all, rows0, rows1, pos_v, sem_g, sem_o):
    nc = 2
    wid = lax.axis_index("s") * nc + lax.axis_index("c")
    base = wid * (P * L)
    rows = (rows0, rows1)
    pltpu.sync_copy(pos_hbm, pos_v)
    pltpu.sync_copy(idx_hbm.at[pl.ds(base, P * L)], idx_all)

    def gather(p, b):
        return pltpu.make_async_copy(
            table_hbm.at[idx_all.at[pl.ds(p * L, L)]], rows[b], sem_g.at[b])

    def out_copy(p, b):
        return pltpu.make_async_copy(
            rows[b], out_hbm.at[pl.ds(base + p * L, L)], sem_o.at[b])

    gather(0, 0).start()

    def step(p0, carry):
        for j in range(2):
            p = p0 + j
            b = j

            @pl.when(p >= 1)
            def _():
                out_copy(p - 1, 1 - b).wait()

            @pl.when(p + 1 < P)
            def _():
                gather(p + 1, 1 - b).start()

            gather(p, b).wait()

            def add_rows(l2, c):
                for r in range(2):
                    for h in range(H // 16):
                        s = pl.ds(h * 16, 16)
                        rows[b][l2 * 2 + r, s] = (
                            rows[b][l2 * 2 + r, s] + pos_v[l2 * 2 + r, s])
                return c

            lax.fori_loop(0, L // 2, add_rows, 0)
            out_copy(p, b).start()
        return carry

    lax.fori_loop(0, P // 2, lambda i, c: step(i * 2, c), 0)
    out_copy(P - 1, (P - 1) % 2).wait()


def kernel(tokens, emb_table):
    B, L = tokens.shape
    V, H = emb_table.shape
    NW = 32
    N = B * L
    assert N % NW == 0 and (N // NW) % L == 0 and H % 16 == 0
    P = (N // NW) // L
    assert P % 2 == 0 and L % 2 == 0

    pos = _make_pos(L, H)
    idx_flat = tokens.reshape(N).astype(jnp.int32)

    mesh = plsc.VectorSubcoreMesh(core_axis_name="c", subcore_axis_name="s")
    sc_k = functools.partial(
        pl.kernel,
        out_type=jax.ShapeDtypeStruct((N, H), jnp.float32),
        mesh=mesh,
        scratch_types=[
            pltpu.VMEM((P * L,), jnp.int32),
            pltpu.VMEM((L, H), jnp.float32),
            pltpu.VMEM((L, H), jnp.float32),
            pltpu.VMEM((L, H), jnp.float32),
            pltpu.SemaphoreType.DMA((2,)),
            pltpu.SemaphoreType.DMA((2,)),
        ],
    )(functools.partial(_sc_body, P, L, H))

    out = sc_k(idx_flat, emb_table, pos)
    return out.reshape(B, L, H)


# speedup vs baseline: 1.7606x; 1.0013x over previous
import functools
import jax
import jax.numpy as jnp
from jax import lax
from jax.experimental import pallas as pl
from jax.experimental.pallas import tpu as pltpu
from jax.experimental.pallas import tpu_sc as plsc

HIDDEN = 128


def _pos_body(out_ref):
    L, H = out_ref.shape
    half = H // 2
    pos = lax.broadcasted_iota(jnp.int32, (L, H), 0).astype(jnp.float32)
    col = lax.broadcasted_iota(jnp.int32, (L, H), 1)
    k = jnp.where(col < half, col, col - half).astype(jnp.float32)
    inv_freq = jnp.exp(k * (-2.0 / H * 9.210340371976184))
    angle = pos * inv_freq
    out_ref[...] = jnp.where(col < half, jnp.sin(angle), jnp.cos(angle))


def _make_pos(L, H):
    return pl.pallas_call(
        _pos_body,
        out_shape=jax.ShapeDtypeStruct((L, H), jnp.float32),
    )()


def _sc_body(P, L, H, idx_hbm, table_hbm, pos_hbm, out_hbm,
             idx_all, rows0, rows1, pos_v, sem_g, sem_o):
    nc = 2
    wid = lax.axis_index("s") * nc + lax.axis_index("c")
    base = wid * (P * L)
    rows = (rows0, rows1)
    pltpu.sync_copy(pos_hbm, pos_v)
    pltpu.sync_copy(idx_hbm.at[pl.ds(base, P * L)], idx_all)

    def gather(p, b):
        return pltpu.make_async_copy(
            table_hbm.at[idx_all.at[pl.ds(p * L, L)]], rows[b], sem_g.at[b])

    def out_copy(p, b):
        return pltpu.make_async_copy(
            rows[b], out_hbm.at[pl.ds(base + p * L, L)], sem_o.at[b])

    gather(0, 0).start()

    def step(p0, carry):
        for j in range(2):
            p = p0 + j
            b = j

            @pl.when(p >= 1)
            def _():
                out_copy(p - 1, 1 - b).wait()

            @pl.when(p + 1 < P)
            def _():
                gather(p + 1, 1 - b).start()

            gather(p, b).wait()

            def add_rows(l2, c):
                for r in range(2):
                    for h in range(H // 16):
                        s = pl.ds(h * 16, 16)
                        plsc.addupdate(rows[b].at[l2 * 2 + r, s],
                                       pos_v[l2 * 2 + r, s])
                return c

            lax.fori_loop(0, L // 2, add_rows, 0)
            out_copy(p, b).start()
        return carry

    lax.fori_loop(0, P // 2, lambda i, c: step(i * 2, c), 0)
    out_copy(P - 1, (P - 1) % 2).wait()


def kernel(tokens, emb_table):
    B, L = tokens.shape
    V, H = emb_table.shape
    NW = 32
    N = B * L
    assert N % NW == 0 and (N // NW) % L == 0 and H % 16 == 0
    P = (N // NW) // L
    assert P % 2 == 0 and L % 2 == 0

    pos = _make_pos(L, H)
    idx_flat = tokens.reshape(N).astype(jnp.int32)

    mesh = plsc.VectorSubcoreMesh(core_axis_name="c", subcore_axis_name="s")
    sc_k = functools.partial(
        pl.kernel,
        out_type=jax.ShapeDtypeStruct((N, H), jnp.float32),
        mesh=mesh,
        scratch_types=[
            pltpu.VMEM((P * L,), jnp.int32),
            pltpu.VMEM((L, H), jnp.float32),
            pltpu.VMEM((L, H), jnp.float32),
            pltpu.VMEM((L, H), jnp.float32),
            pltpu.SemaphoreType.DMA((2,)),
            pltpu.SemaphoreType.DMA((2,)),
        ],
    )(functools.partial(_sc_body, P, L, H))

    out = sc_k(idx_flat, emb_table, pos)
    return out.reshape(B, L, H)


# speedup vs baseline: 2.1176x; 1.2028x over previous
import functools
import jax
import jax.numpy as jnp
from jax import lax
from jax.experimental import pallas as pl
from jax.experimental.pallas import tpu as pltpu
from jax.experimental.pallas import tpu_sc as plsc

HIDDEN = 128


def _pos_body(out_ref):
    L, H = out_ref.shape
    half = H // 2
    pos = lax.broadcasted_iota(jnp.int32, (L, H), 0).astype(jnp.float32)
    col = lax.broadcasted_iota(jnp.int32, (L, H), 1)
    k = jnp.where(col < half, col, col - half).astype(jnp.float32)
    inv_freq = jnp.exp(k * (-2.0 / H * 9.210340371976184))
    angle = pos * inv_freq
    out_ref[...] = jnp.where(col < half, jnp.sin(angle), jnp.cos(angle))


def _make_pos(L, H):
    return pl.pallas_call(
        _pos_body,
        out_shape=jax.ShapeDtypeStruct((L, H), jnp.float32),
    )()


def _sc_body(P, L, H, idx_hbm, table_hbm, pos_hbm, out_hbm,
             idx_all, rows0, rows1, rows2, pos_v, sem_g, sem_o):
    nc = 2
    wid = lax.axis_index("s") * nc + lax.axis_index("c")
    base = wid * (P * L)
    rows = (rows0, rows1, rows2)
    NB = 3
    pltpu.sync_copy(pos_hbm, pos_v)
    pltpu.sync_copy(idx_hbm.at[pl.ds(base, P * L)], idx_all)

    def gather(p, b):
        return pltpu.make_async_copy(
            table_hbm.at[idx_all.at[pl.ds(p * L, L)]], rows[b], sem_g.at[b])

    def out_copy(p, b):
        return pltpu.make_async_copy(
            rows[b], out_hbm.at[pl.ds(base + p * L, L)], sem_o.at[b])

    def add_pos(b):
        def add_rows(l2, c):
            for r in range(2):
                for h in range(H // 16):
                    s = pl.ds(h * 16, 16)
                    plsc.addupdate(rows[b].at[l2 * 2 + r, s],
                                   pos_v[l2 * 2 + r, s])
            return c

        lax.fori_loop(0, L // 2, add_rows, 0)

    def one_period(p, b, wait_prev, start_next):
        bn = (b + 1) % NB
        if wait_prev:
            out_copy(p - 2, bn).wait()
        if start_next:
            gather(p + 1, bn).start()
        gather(p, b).wait()
        add_pos(b)
        out_copy(p, b).start()

    gather(0, 0).start()

    for p in range(NB):
        one_period(p, p % NB, p >= 2, p + 1 < P)

    n_steady = (P - NB) // NB - 1

    def step(i, carry):
        p0 = NB + i * NB
        for j in range(NB):
            one_period(p0 + j, j, True, True)
        return carry

    lax.fori_loop(0, n_steady, step, 0)

    for p in range(NB + n_steady * NB, P):
        one_period(p, p % NB, True, p + 1 < P)

    out_copy(P - 2, (P - 2) % NB).wait()
    out_copy(P - 1, (P - 1) % NB).wait()


def kernel(tokens, emb_table):
    B, L = tokens.shape
    V, H = emb_table.shape
    NW = 32
    N = B * L
    assert N % NW == 0 and (N // NW) % L == 0 and H % 16 == 0
    P = (N // NW) // L
    assert P % 2 == 0 and L % 2 == 0

    pos = _make_pos(L, H)
    idx_flat = tokens.reshape(N).astype(jnp.int32)

    mesh = plsc.VectorSubcoreMesh(core_axis_name="c", subcore_axis_name="s")
    sc_k = functools.partial(
        pl.kernel,
        out_type=jax.ShapeDtypeStruct((N, H), jnp.float32),
        mesh=mesh,
        scratch_types=[
            pltpu.VMEM((P * L,), jnp.int32),
            pltpu.VMEM((L, H), jnp.float32),
            pltpu.VMEM((L, H), jnp.float32),
            pltpu.VMEM((L, H), jnp.float32),
            pltpu.VMEM((L, H), jnp.float32),
            pltpu.SemaphoreType.DMA((3,)),
            pltpu.SemaphoreType.DMA((3,)),
        ],
    )(functools.partial(_sc_body, P, L, H))

    out = sc_k(idx_flat, emb_table, pos)
    return out.reshape(B, L, H)


# speedup vs baseline: 2.1230x; 1.0026x over previous
import functools
import jax
import jax.numpy as jnp
from jax import lax
from jax.experimental import pallas as pl
from jax.experimental.pallas import tpu as pltpu
from jax.experimental.pallas import tpu_sc as plsc

HIDDEN = 128


def _pos_body(out_ref):
    L, H = out_ref.shape
    half = H // 2
    pos = lax.broadcasted_iota(jnp.int32, (L, H), 0).astype(jnp.float32)
    col = lax.broadcasted_iota(jnp.int32, (L, H), 1)
    k = jnp.where(col < half, col, col - half).astype(jnp.float32)
    inv_freq = jnp.exp(k * (-2.0 / H * 9.210340371976184))
    angle = pos * inv_freq
    out_ref[...] = jnp.where(col < half, jnp.sin(angle), jnp.cos(angle))


def _make_pos(L, H):
    return pl.pallas_call(
        _pos_body,
        out_shape=jax.ShapeDtypeStruct((L, H), jnp.float32),
    )()


def _sc_body(P, L, H, idx_hbm, table_hbm, pos_hbm, out_hbm,
             idx_all, rows0, rows1, rows2, pos_v, sem_g, sem_o, sem_s):
    nc = 2
    wid = lax.axis_index("s") * nc + lax.axis_index("c")
    base = wid * (P * L)
    rows = (rows0, rows1, rows2)
    NB = 3

    def gather(p, b):
        return pltpu.make_async_copy(
            table_hbm.at[idx_all.at[pl.ds(p * L, L)]], rows[b], sem_g.at[b])

    def out_copy(p, b):
        return pltpu.make_async_copy(
            rows[b], out_hbm.at[pl.ds(base + p * L, L)], sem_o.at[b])

    def add_pos(b):
        def add_rows(l2, c):
            for r in range(2):
                for h in range(H // 16):
                    s = pl.ds(h * 16, 16)
                    plsc.addupdate(rows[b].at[l2 * 2 + r, s],
                                   pos_v[l2 * 2 + r, s])
            return c

        lax.fori_loop(0, L // 2, add_rows, 0)

    idx_head = pltpu.make_async_copy(
        idx_hbm.at[pl.ds(base, L)], idx_all.at[pl.ds(0, L)], sem_s.at[0])
    idx_tail = pltpu.make_async_copy(
        idx_hbm.at[pl.ds(base + L, (P - 1) * L)],
        idx_all.at[pl.ds(L, (P - 1) * L)], sem_s.at[1])
    pos_copy = pltpu.make_async_copy(pos_hbm, pos_v, sem_s.at[2])
    idx_head.start()
    idx_tail.start()
    pos_copy.start()

    def one_period(p, b, wait_prev, start_next):
        bn = (b + 1) % NB
        if wait_prev:
            out_copy(p - 2, bn).wait()
        if start_next:
            gather(p + 1, bn).start()
        gather(p, b).wait()
        add_pos(b)
        out_copy(p, b).start()

    idx_head.wait()
    gather(0, 0).start()
    idx_tail.wait()
    pos_copy.wait()

    for p in range(NB):
        one_period(p, p % NB, p >= 2, p + 1 < P)

    n_steady = (P - NB) // NB - 1

    def step(i, carry):
        p0 = NB + i * NB
        for j in range(NB):
            one_period(p0 + j, j, True, True)
        return carry

    lax.fori_loop(0, n_steady, step, 0)

    for p in range(NB + n_steady * NB, P):
        one_period(p, p % NB, True, p + 1 < P)

    out_copy(P - 2, (P - 2) % NB).wait()
    out_copy(P - 1, (P - 1) % NB).wait()


def kernel(tokens, emb_table):
    B, L = tokens.shape
    V, H = emb_table.shape
    NW = 32
    N = B * L
    assert N % NW == 0 and (N // NW) % L == 0 and H % 16 == 0
    P = (N // NW) // L
    assert P % 2 == 0 and L % 2 == 0

    pos = _make_pos(L, H)
    idx_flat = tokens.reshape(N).astype(jnp.int32)

    mesh = plsc.VectorSubcoreMesh(core_axis_name="c", subcore_axis_name="s")
    sc_k = functools.partial(
        pl.kernel,
        out_type=jax.ShapeDtypeStruct((N, H), jnp.float32),
        mesh=mesh,
        scratch_types=[
            pltpu.VMEM((P * L,), jnp.int32),
            pltpu.VMEM((L, H), jnp.float32),
            pltpu.VMEM((L, H), jnp.float32),
            pltpu.VMEM((L, H), jnp.float32),
            pltpu.VMEM((L, H), jnp.float32),
            pltpu.SemaphoreType.DMA((3,)),
            pltpu.SemaphoreType.DMA((3,)),
            pltpu.SemaphoreType.DMA((3,)),
        ],
    )(functools.partial(_sc_body, P, L, H))

    out = sc_k(idx_flat, emb_table, pos)
    return out.reshape(B, L, H)


# speedup vs baseline: 2.1245x; 1.0007x over previous
import functools
import jax
import jax.numpy as jnp
import numpy as np
from jax import lax
from jax.experimental import pallas as pl
from jax.experimental.pallas import tpu as pltpu
from jax.experimental.pallas import tpu_sc as plsc

HIDDEN = 128


def _make_pos(L, H):
    pos = np.arange(L, dtype=np.float32)
    inv_freq = 1.0 / (10000.0 ** (np.arange(0, H, 2, dtype=np.float32) / H))
    ang = pos[:, None] * inv_freq[None, :]
    return jnp.asarray(np.concatenate([np.sin(ang), np.cos(ang)], axis=-1))


def _sc_body(P, L, H, idx_hbm, table_hbm, pos_hbm, out_hbm,
             idx_all, rows0, rows1, rows2, pos_v, sem_g, sem_o, sem_s):
    nc = 2
    wid = lax.axis_index("s") * nc + lax.axis_index("c")
    base = wid * (P * L)
    rows = (rows0, rows1, rows2)
    NB = 3

    def gather(p, b):
        return pltpu.make_async_copy(
            table_hbm.at[idx_all.at[pl.ds(p * L, L)]], rows[b], sem_g.at[b])

    def out_copy(p, b):
        return pltpu.make_async_copy(
            rows[b], out_hbm.at[pl.ds(base + p * L, L)], sem_o.at[b])

    def add_pos(b):
        def add_rows(l2, c):
            for r in range(2):
                for h in range(H // 16):
                    s = pl.ds(h * 16, 16)
                    plsc.addupdate(rows[b].at[l2 * 2 + r, s],
                                   pos_v[l2 * 2 + r, s])
            return c

        lax.fori_loop(0, L // 2, add_rows, 0)

    idx_head = pltpu.make_async_copy(
        idx_hbm.at[pl.ds(base, L)], idx_all.at[pl.ds(0, L)], sem_s.at[0])
    idx_tail = pltpu.make_async_copy(
        idx_hbm.at[pl.ds(base + L, (P - 1) * L)],
        idx_all.at[pl.ds(L, (P - 1) * L)], sem_s.at[1])
    pos_copy = pltpu.make_async_copy(pos_hbm, pos_v, sem_s.at[2])
    idx_head.start()
    idx_tail.start()
    pos_copy.start()

    def one_period(p, b, wait_prev, start_next):
        bn = (b + 1) % NB
        if wait_prev:
            out_copy(p - 2, bn).wait()
        if start_next:
            gather(p + 1, bn).start()
        gather(p, b).wait()
        add_pos(b)
        out_copy(p, b).start()

    idx_head.wait()
    gather(0, 0).start()
    idx_tail.wait()
    pos_copy.wait()

    for p in range(NB):
        one_period(p, p % NB, p >= 2, p + 1 < P)

    n_steady = (P - NB) // NB - 1

    def step(i, carry):
        p0 = NB + i * NB
        for j in range(NB):
            one_period(p0 + j, j, True, True)
        return carry

    lax.fori_loop(0, n_steady, step, 0)

    for p in range(NB + n_steady * NB, P):
        one_period(p, p % NB, True, p + 1 < P)

    out_copy(P - 2, (P - 2) % NB).wait()
    out_copy(P - 1, (P - 1) % NB).wait()


def kernel(tokens, emb_table):
    B, L = tokens.shape
    V, H = emb_table.shape
    NW = 32
    N = B * L
    assert N % NW == 0 and (N // NW) % L == 0 and H % 16 == 0
    P = (N // NW) // L
    assert P % 2 == 0 and L % 2 == 0

    pos = _make_pos(L, H)
    idx_flat = tokens.reshape(N).astype(jnp.int32)

    mesh = plsc.VectorSubcoreMesh(core_axis_name="c", subcore_axis_name="s")
    sc_k = functools.partial(
        pl.kernel,
        out_type=jax.ShapeDtypeStruct((N, H), jnp.float32),
        mesh=mesh,
        scratch_types=[
            pltpu.VMEM((P * L,), jnp.int32),
            pltpu.VMEM((L, H), jnp.float32),
            pltpu.VMEM((L, H), jnp.float32),
            pltpu.VMEM((L, H), jnp.float32),
            pltpu.VMEM((L, H), jnp.float32),
            pltpu.SemaphoreType.DMA((3,)),
            pltpu.SemaphoreType.DMA((3,)),
            pltpu.SemaphoreType.DMA((3,)),
        ],
    )(functools.partial(_sc_body, P, L, H))

    out = sc_k(idx_flat, emb_table, pos)
    return out.reshape(B, L, H)
